# baseline (speedup 1.0000x reference)
import functools

import jax
import jax.numpy as jnp
from jax import lax
from jax.experimental import pallas as pl
from jax.experimental.pallas import tpu as pltpu
from jax.experimental.pallas import tpu_sc as plsc

NUM_CORES = 2
NUM_SUBCORES = 16
NUM_WORKERS = NUM_CORES * NUM_SUBCORES
LANES = 16


def _build(B, D):
    b_per_w = B // NUM_WORKERS
    mesh = plsc.VectorSubcoreMesh(
        core_axis_name="c", subcore_axis_name="s", num_cores=NUM_CORES
    )

    @functools.partial(
        pl.kernel,
        out_type=jax.ShapeDtypeStruct((B,), jnp.float32),
        mesh=mesh,
        compiler_params=pltpu.CompilerParams(
            needs_layout_passes=False, use_tc_tiling_on_sc=False),
        scratch_types=[
            pltpu.VMEM((b_per_w,), jnp.int32),
            pltpu.VMEM((b_per_w,), jnp.int32),
            pltpu.VMEM((b_per_w, D), jnp.float32),
            pltpu.VMEM((b_per_w, D), jnp.float32),
            pltpu.VMEM((b_per_w,), jnp.float32),
            pltpu.VMEM((b_per_w,), jnp.float32),
            pltpu.VMEM((LANES,), jnp.float32),
            pltpu.VMEM((b_per_w,), jnp.float32),
            pltpu.SemaphoreType.DMA,
        ],
    )
    def mf_kernel(user_hbm, item_hbm, uemb_hbm, iemb_hbm, ubias_hbm,
                  ibias_hbm, gbias_hbm, out_hbm,
                  uidx_v, iidx_v, urows_v, irows_v, ubias_v, ibias_v,
                  gbias_v, out_v, sem):
        wid = lax.axis_index("s") * NUM_CORES + lax.axis_index("c")
        base = wid * b_per_w

        pltpu.sync_copy(user_hbm.at[pl.ds(base, b_per_w)], uidx_v)
        pltpu.sync_copy(item_hbm.at[pl.ds(base, b_per_w)], iidx_v)
        pltpu.sync_copy(gbias_hbm, gbias_v)

        cu = pltpu.async_copy(uemb_hbm.at[uidx_v], urows_v, sem)
        ci = pltpu.async_copy(iemb_hbm.at[iidx_v], irows_v, sem)
        cbu = pltpu.async_copy(ubias_hbm.at[uidx_v], ubias_v, sem)
        cbi = pltpu.async_copy(ibias_hbm.at[iidx_v], ibias_v, sem)
        cu.wait()
        ci.wait()
        cbu.wait()
        cbi.wait()

        iota16 = lax.iota(jnp.int32, LANES)
        gsplat = gbias_v[...]

        def group_body(g, carry):
            rbase = g * LANES
            rows = rbase + iota16
            acc = gsplat + ubias_v[pl.ds(rbase, LANES)] + ibias_v[pl.ds(rbase, LANES)]

            def dim_body(d, acc):
                cols = jnp.full((LANES,), 0, jnp.int32) + d
                uv = plsc.load_gather(urows_v, [rows, cols])
                iv = plsc.load_gather(irows_v, [rows, cols])
                return acc + uv * iv

            acc = lax.fori_loop(0, D, dim_body, acc)
            out_v[pl.ds(rbase, LANES)] = acc
            return carry

        lax.fori_loop(0, b_per_w // LANES, group_body, 0)
        pltpu.sync_copy(out_v, out_hbm.at[pl.ds(base, b_per_w)])

    return mf_kernel


def kernel(user, item, user_emb, item_emb, user_bias, item_bias, global_bias):
    B = user.shape[0]
    D = user_emb.shape[1]
    mf = _build(B, D)
    gb16 = jnp.broadcast_to(global_bias.reshape(()), (LANES,))
    return mf(user.astype(jnp.int32), item.astype(jnp.int32),
              user_emb, item_emb,
              user_bias.reshape(-1), item_bias.reshape(-1), gb16)
